# Initial kernel scaffold; baseline (speedup 1.0000x reference)
import functools

import jax
import jax.numpy as jnp
from jax import lax
from jax.experimental import pallas as pl
from jax.experimental.pallas import tpu as pltpu
from jax.experimental.pallas import tpu_sc as plsc

N = 10000
E = 320000
D = 128
NC = 3
NTILES = 16
W = 80
EPT = E // NTILES
CH = EPT // W
NROW = N // NTILES

_mesh = plsc.VectorSubcoreMesh(core_axis_name="c", subcore_axis_name="s")


def _seg_body(table, src_idx, dst_idx, zeros128, zeros16, ones16,
              out_seg, out_cnt, srcv, dstv, rows, ones, accum, cacc,
              with_counts):
    cid = lax.axis_index("c")
    sid = lax.axis_index("s")
    r0 = sid * NROW
    pltpu.sync_copy(zeros128.at[pl.ds(r0, NROW)], accum.at[pl.ds(r0, NROW)])
    if with_counts:
        pltpu.sync_copy(zeros16.at[pl.ds(r0, NROW)], cacc.at[pl.ds(r0, NROW)])
        pltpu.sync_copy(ones16, ones)
    pltpu.sync_copy(src_idx.at[cid, sid], srcv)
    pltpu.sync_copy(dst_idx.at[cid, sid], dstv)
    plsc.subcore_barrier()

    def body(j, _):
        pltpu.sync_copy(table.at[srcv.at[j]], rows)
        pltpu.sync_copy(rows, accum.at[dstv.at[j]], add=True)
        if with_counts:
            pltpu.sync_copy(ones, cacc.at[dstv.at[j]], add=True)
        return 0

    lax.fori_loop(0, CH, body, 0)
    plsc.subcore_barrier()
    pltpu.sync_copy(accum.at[pl.ds(r0, NROW)],
                    out_seg.at[cid, pl.ds(r0, NROW)])
    if with_counts:
        pltpu.sync_copy(cacc.at[pl.ds(r0, NROW)],
                        out_cnt.at[cid, pl.ds(r0, NROW)])


@functools.partial(
    pl.kernel, mesh=_mesh,
    out_type=(jax.ShapeDtypeStruct((2, N, D), jnp.float32),
              jax.ShapeDtypeStruct((2, N, 16), jnp.float32)),
    scratch_types=[
        pltpu.VMEM((CH, W), jnp.int32),
        pltpu.VMEM((CH, W), jnp.int32),
        pltpu.VMEM((W, D), jnp.float32),
        pltpu.VMEM((W, 16), jnp.float32),
        pltpu.VMEM_SHARED((N, D), jnp.float32),
        pltpu.VMEM_SHARED((N, 16), jnp.float32),
    ],
)
def _seg_counts_kernel(table, src_idx, dst_idx, zeros128, zeros16, ones16,
                       out_seg, out_cnt, srcv, dstv, rows, ones, accum, cacc):
    _seg_body(table, src_idx, dst_idx, zeros128, zeros16, ones16,
              out_seg, out_cnt, srcv, dstv, rows, ones, accum, cacc, True)


@functools.partial(
    pl.kernel, mesh=_mesh,
    out_type=jax.ShapeDtypeStruct((2, N, D), jnp.float32),
    scratch_types=[
        pltpu.VMEM((CH, W), jnp.int32),
        pltpu.VMEM((CH, W), jnp.int32),
        pltpu.VMEM((W, D), jnp.float32),
        pltpu.VMEM_SHARED((N, D), jnp.float32),
    ],
)
def _seg_kernel(table, src_idx, dst_idx, zeros128,
                out_seg, srcv, dstv, rows, accum):
    _seg_body(table, src_idx, dst_idx, zeros128, None, None,
              out_seg, None, srcv, dstv, rows, None, accum, None, False)


@functools.partial(
    pl.kernel, mesh=_mesh,
    out_type=jax.ShapeDtypeStruct((2, E, D), jnp.float32),
    scratch_types=[
        pltpu.VMEM((CH, W), jnp.int32),
        pltpu.VMEM((W, D), jnp.float32),
        pltpu.SemaphoreType.DMA,
    ],
)
def _edge_gather_kernel(table, idx, out, idxv, rows, sem):
    cid = lax.axis_index("c")
    sid = lax.axis_index("s")
    pltpu.sync_copy(idx.at[cid, sid], idxv)
    base = sid * EPT

    def body(j, _):
        pltpu.async_copy(table.at[idxv.at[j]], rows, sem).wait()
        pltpu.sync_copy(rows, out.at[cid, pl.ds(base + j * W, W)])
        return 0

    lax.fori_loop(0, CH, body, 0)


def _proj_tc(x_ref, w_ref, b_ref, o_ref):
    o_ref[...] = jax.nn.relu(
        jnp.dot(x_ref[0], w_ref[0], preferred_element_type=jnp.float32)
        + b_ref[0][None, :])[None]


def _combine_tc(seg_ref, cnt_ref, xd_ref, wl_ref, bl_ref, wr_ref, o_ref):
    cnt = cnt_ref[0][:, 0:1]
    mean = seg_ref[0] / jnp.maximum(cnt, 1.0)
    o = (jnp.dot(mean, wl_ref[0], preferred_element_type=jnp.float32)
         + bl_ref[0][None, :]
         + jnp.dot(xd_ref[0], wr_ref[0], preferred_element_type=jnp.float32))
    o_ref[...] = jax.nn.relu(o)[None]


def _combine2_tc(seg_ref, cnt_ref, xd_ref, wl_ref, bl_ref, wr_ref, w2_ref,
                 o_ref):
    cnt = cnt_ref[0][:, 0:1]
    mean = seg_ref[0] / jnp.maximum(cnt, 1.0)
    o = (jnp.dot(mean, wl_ref[0], preferred_element_type=jnp.float32)
         + bl_ref[0][None, :]
         + jnp.dot(xd_ref[0], wr_ref[0], preferred_element_type=jnp.float32))
    o_ref[...] = jnp.dot(jax.nn.relu(o), w2_ref[0],
                         preferred_element_type=jnp.float32)[None]


def _decoder_tc(g_ref, attr_ref, w1c_ref, b1_ref, w2_ref, b2_ref, o_ref):
    zh = (g_ref[0] + g_ref[1]
          + jnp.dot(attr_ref[...], w1c_ref[...],
                    preferred_element_type=jnp.float32)
          + b1_ref[0][None, :])
    o_ref[...] = (jnp.dot(jax.nn.relu(zh), w2_ref[...],
                          preferred_element_type=jnp.float32)
                  + b2_ref[0][None, :])


_RB = 2000
_EB = 4000


def _proj(xs, ws, bs):
    return pl.pallas_call(
        _proj_tc,
        grid=(2, N // _RB),
        in_specs=[
            pl.BlockSpec((1, _RB, D), lambda t, k: (t, k, 0)),
            pl.BlockSpec((1, D, D), lambda t, k: (t, 0, 0)),
            pl.BlockSpec((1, D), lambda t, k: (t, 0)),
        ],
        out_specs=pl.BlockSpec((1, _RB, D), lambda t, k: (t, k, 0)),
        out_shape=jax.ShapeDtypeStruct((2, N, D), jnp.float32),
    )(xs, ws, bs)


def _combine(seg, cnt, xd, wl, bl, wr, swap_xd):
    xmap = (lambda t, k: (1 - t, k, 0)) if swap_xd else (lambda t, k: (t, k, 0))
    return pl.pallas_call(
        _combine_tc,
        grid=(2, N // _RB),
        in_specs=[
            pl.BlockSpec((1, _RB, D), lambda t, k: (t, k, 0)),
            pl.BlockSpec((1, _RB, 16), lambda t, k: (t, k, 0)),
            pl.BlockSpec((1, _RB, D), xmap),
            pl.BlockSpec((1, D, D), lambda t, k: (t, 0, 0)),
            pl.BlockSpec((1, D), lambda t, k: (t, 0)),
            pl.BlockSpec((1, D, D), lambda t, k: (t, 0, 0)),
        ],
        out_specs=pl.BlockSpec((1, _RB, D), lambda t, k: (t, k, 0)),
        out_shape=jax.ShapeDtypeStruct((2, N, D), jnp.float32),
    )(seg, cnt, xd, wl, bl, wr)


def _combine2(seg, cnt, xd, wl, bl, wr, w2):
    return pl.pallas_call(
        _combine2_tc,
        grid=(2, N // _RB),
        in_specs=[
            pl.BlockSpec((1, _RB, D), lambda t, k: (t, k, 0)),
            pl.BlockSpec((1, _RB, 16), lambda t, k: (t, k, 0)),
            pl.BlockSpec((1, _RB, D), lambda t, k: (t, k, 0)),
            pl.BlockSpec((1, D, D), lambda t, k: (t, 0, 0)),
            pl.BlockSpec((1, D), lambda t, k: (t, 0)),
            pl.BlockSpec((1, D, D), lambda t, k: (t, 0, 0)),
            pl.BlockSpec((1, D, D), lambda t, k: (t, 0, 0)),
        ],
        out_specs=pl.BlockSpec((1, _RB, D), lambda t, k: (t, k, 0)),
        out_shape=jax.ShapeDtypeStruct((2, N, D), jnp.float32),
    )(seg, cnt, xd, wl, bl, wr, w2)


def _decoder(g, attr, w1c, b1, w2, b2):
    return pl.pallas_call(
        _decoder_tc,
        grid=(E // _EB,),
        in_specs=[
            pl.BlockSpec((2, _EB, D), lambda k: (0, k, 0)),
            pl.BlockSpec((_EB, 2), lambda k: (k, 0)),
            pl.BlockSpec((2, D), lambda k: (0, 0)),
            pl.BlockSpec((1, D), lambda k: (0, 0)),
            pl.BlockSpec((D, NC), lambda k: (0, 0)),
            pl.BlockSpec((1, NC), lambda k: (0, 0)),
        ],
        out_specs=pl.BlockSpec((_EB, NC), lambda k: (k, 0)),
        out_shape=jax.ShapeDtypeStruct((E, NC), jnp.float32),
    )(g, attr, w1c, b1, w2, b2)


def _tile_idx(a):
    return a.astype(jnp.int32).reshape(NTILES, CH, W)


def kernel(x_politician, x_stock, edge_index_trades, edge_index_rev,
           trade_edge_index, trade_edge_attr, lin_pol_W, lin_pol_b,
           lin_stock_W, lin_stock_b,
           c1t_Wl, c1t_bl, c1t_Wr, c1r_Wl, c1r_bl, c1r_Wr,
           c2t_Wl, c2t_bl, c2t_Wr, c2r_Wl, c2r_bl, c2r_Wr,
           dec_W1, dec_b1, dec_W2, dec_b2):
    f32 = jnp.float32
    src_t = edge_index_trades[0].astype(jnp.int32)
    dst_t = edge_index_trades[1].astype(jnp.int32)
    src_r = edge_index_rev[0].astype(jnp.int32)
    dst_r = edge_index_rev[1].astype(jnp.int32)
    row_d = trade_edge_index[0].astype(jnp.int32)
    col_d = trade_edge_index[1].astype(jnp.int32)

    zeros128 = jnp.zeros((N, D), f32)
    zeros16 = jnp.zeros((N, 16), f32)
    ones16 = jnp.ones((W, 16), f32)

    dst_idx = jnp.stack([_tile_idx(dst_t), _tile_idx(dst_r)])
    src1 = jnp.stack([_tile_idx(src_t), _tile_idx(src_r + N)])
    src2 = jnp.stack([_tile_idx(src_t + N), _tile_idx(src_r)])
    dec_idx = jnp.stack([_tile_idx(row_d + N), _tile_idx(col_d)])

    xs = jnp.stack([x_politician, x_stock])
    ws = jnp.stack([lin_pol_W, lin_stock_W])
    bs = jnp.stack([lin_pol_b, lin_stock_b])
    h0 = _proj(xs, ws, bs)

    seg1, cnt = _seg_counts_kernel(h0.reshape(2 * N, D), src1, dst_idx,
                                   zeros128, zeros16, ones16)
    h1 = _combine(seg1, cnt,
                  h0,
                  jnp.stack([c1t_Wl, c1r_Wl]),
                  jnp.stack([c1t_bl, c1r_bl]),
                  jnp.stack([c1t_Wr, c1r_Wr]),
                  swap_xd=True)

    seg2 = _seg_kernel(h1.reshape(2 * N, D), src2, dst_idx, zeros128)
    w1_stock = dec_W1[D:2 * D]
    w1_pol = dec_W1[:D]
    ps = _combine2(seg2, cnt,
                   h1,
                   jnp.stack([c2t_Wl, c2r_Wl]),
                   jnp.stack([c2t_bl, c2r_bl]),
                   jnp.stack([c2t_Wr, c2r_Wr]),
                   jnp.stack([w1_stock, w1_pol]))

    g = _edge_gather_kernel(ps.reshape(2 * N, D), dec_idx)
    w1c = dec_W1[2 * D:]
    return _decoder(g, trade_edge_attr, w1c, dec_b1[None], dec_W2,
                    dec_b2[None])


# speedup vs baseline: 1.9332x; 1.9332x over previous
import functools

import jax
import jax.numpy as jnp
from jax import lax
from jax.experimental import pallas as pl
from jax.experimental.pallas import tpu as pltpu
from jax.experimental.pallas import tpu_sc as plsc

N = 10000
E = 320000
D = 128
HD = D // 2
NC = 3
NTILES = 16
W = 80
EPT = E // NTILES
CH = EPT // W
NP = 10240
NROW = NP // NTILES


def _seg_phases(table, src_idx, dst_idx, zeros64, out_seg,
                srcv, dstv, rows, accum, cid, sid, extra_phase0=None,
                extra_loop0=None):
    r0 = sid * NROW
    for rel in (0, 1):
        pltpu.sync_copy(zeros64.at[pl.ds(r0, NROW)],
                        accum.at[pl.ds(r0, NROW)])
        pltpu.sync_copy(src_idx.at[cid, rel, sid], srcv)
        pltpu.sync_copy(dst_idx.at[rel, sid], dstv)
        if rel == 0 and extra_phase0 is not None:
            extra_phase0()
        plsc.subcore_barrier()

        def body(j, _):
            pltpu.sync_copy(table.at[srcv.at[j]], rows)
            pltpu.sync_copy(rows, accum.at[dstv.at[j]], add=True)
            if rel == 0 and extra_loop0 is not None:
                extra_loop0(j)
            return 0

        lax.fori_loop(0, CH, body, 0)
        plsc.subcore_barrier()
        pltpu.sync_copy(accum.at[pl.ds(r0, NROW)],
                        out_seg.at[rel, cid, pl.ds(r0, NROW)])


@functools.lru_cache(maxsize=None)
def _sc_kernels():
    mesh = plsc.VectorSubcoreMesh(core_axis_name="c", subcore_axis_name="s")
    cp = pltpu.CompilerParams(use_tc_tiling_on_sc=False)

    @functools.partial(
        pl.kernel, mesh=mesh, compiler_params=cp,
        out_type=(jax.ShapeDtypeStruct((2, 2, NP, HD), jnp.float32),
                  jax.ShapeDtypeStruct((2, NP, 16), jnp.float32)),
        scratch_types=[
            pltpu.VMEM((CH, W), jnp.int32),
            pltpu.VMEM((CH, W), jnp.int32),
            pltpu.VMEM((CH, W), jnp.int32),
            pltpu.VMEM((W, HD), jnp.float32),
            pltpu.VMEM((W, 16), jnp.float32),
            pltpu.VMEM_SHARED((NP, HD), jnp.float32),
            pltpu.VMEM_SHARED((NP, 16), jnp.float32),
        ],
    )
    def seg_counts_kernel(table, src_idx, dst_idx, zeros64, zeros16, ones16,
                          out_seg, out_cnt, srcv, dstv, cntv, rows, ones,
                          accum, cacc):
        cid = lax.axis_index("c")
        sid = lax.axis_index("s")
        r0 = sid * NROW

        def phase0_setup():
            pltpu.sync_copy(zeros16.at[pl.ds(r0, NROW)],
                            cacc.at[pl.ds(r0, NROW)])
            pltpu.sync_copy(ones16, ones)
            pltpu.sync_copy(dst_idx.at[cid, sid], cntv)

        def loop0(j):
            pltpu.sync_copy(ones, cacc.at[cntv.at[j]], add=True)

        _seg_phases(table, src_idx, dst_idx, zeros64, out_seg,
                    srcv, dstv, rows, accum, cid, sid,
                    extra_phase0=phase0_setup, extra_loop0=loop0)
        pltpu.sync_copy(cacc.at[pl.ds(r0, NROW)],
                        out_cnt.at[cid, pl.ds(r0, NROW)])

    @functools.partial(
        pl.kernel, mesh=mesh, compiler_params=cp,
        out_type=jax.ShapeDtypeStruct((2, 2, NP, HD), jnp.float32),
        scratch_types=[
            pltpu.VMEM((CH, W), jnp.int32),
            pltpu.VMEM((CH, W), jnp.int32),
            pltpu.VMEM((W, HD), jnp.float32),
            pltpu.VMEM_SHARED((NP, HD), jnp.float32),
        ],
    )
    def seg_kernel(table, src_idx, dst_idx, zeros64,
                   out_seg, srcv, dstv, rows, accum):
        cid = lax.axis_index("c")
        sid = lax.axis_index("s")
        _seg_phases(table, src_idx, dst_idx, zeros64, out_seg,
                    srcv, dstv, rows, accum, cid, sid)

    @functools.partial(
        pl.kernel, mesh=mesh, compiler_params=cp,
        out_type=jax.ShapeDtypeStruct((2, 2, E, HD), jnp.float32),
        scratch_types=[
            pltpu.VMEM((CH, W), jnp.int32),
            pltpu.VMEM((W, HD), jnp.float32),
            pltpu.SemaphoreType.DMA,
        ],
    )
    def edge_gather_kernel(table, idx, out, idxv, rows, sem):
        cid = lax.axis_index("c")
        sid = lax.axis_index("s")
        base = sid * EPT
        for s in (0, 1):
            pltpu.sync_copy(idx.at[cid, s, sid], idxv)

            def body(j, _):
                pltpu.async_copy(table.at[idxv.at[j]], rows, sem).wait()
                pltpu.sync_copy(rows, out.at[s, cid, pl.ds(base + j * W, W)])
                return 0

            lax.fori_loop(0, CH, body, 0)

    return seg_counts_kernel, seg_kernel, edge_gather_kernel


def _split(h):
    return jnp.stack([h[:, :HD], h[:, HD:]])[:, None]


def _join(x):
    return jnp.concatenate([x[0, 0], x[1, 0]], axis=-1)


def _proj_tc(x_ref, w_ref, b_ref, o_ref):
    o_ref[...] = _split(jax.nn.relu(
        jnp.dot(x_ref[0], w_ref[0], preferred_element_type=jnp.float32)
        + b_ref[0]))


def _combine_o(seg_ref, cnt_ref, xd_ref, wl_ref, bl_ref, wr_ref):
    cnt = cnt_ref[0][:, 0:1]
    seg = jnp.concatenate([seg_ref[0, 0], seg_ref[0, 1]], axis=-1)
    mean = seg / jnp.maximum(cnt, 1.0)
    return (jnp.dot(mean, wl_ref[0], preferred_element_type=jnp.float32)
            + bl_ref[0]
            + jnp.dot(_join(xd_ref), wr_ref[0],
                      preferred_element_type=jnp.float32))


def _combine_tc(seg_ref, cnt_ref, xd_ref, wl_ref, bl_ref, wr_ref, o_ref):
    o = _combine_o(seg_ref, cnt_ref, xd_ref, wl_ref, bl_ref, wr_ref)
    o_ref[...] = _split(jax.nn.relu(o))


def _combine2_tc(seg_ref, cnt_ref, xd_ref, wl_ref, bl_ref, wr_ref, w2_ref,
                 o_ref):
    o = _combine_o(seg_ref, cnt_ref, xd_ref, wl_ref, bl_ref, wr_ref)
    o_ref[...] = _split(jnp.dot(jax.nn.relu(o), w2_ref[0],
                                preferred_element_type=jnp.float32))


def _decoder_tc(g_ref, attr_ref, w1c_ref, b1_ref, w2_ref, b2_ref, o_ref):
    zh = (jnp.concatenate([g_ref[0, 0] + g_ref[1, 0],
                           g_ref[0, 1] + g_ref[1, 1]], axis=-1)
          + jnp.dot(attr_ref[...], w1c_ref[...],
                    preferred_element_type=jnp.float32)
          + b1_ref[...])
    o_ref[...] = (jnp.dot(jax.nn.relu(zh), w2_ref[...],
                          preferred_element_type=jnp.float32)
                  + b2_ref[...])


_RB = 2000
_EB = 4000


def _proj(xs, ws, bs):
    return pl.pallas_call(
        _proj_tc,
        grid=(2, N // _RB),
        in_specs=[
            pl.BlockSpec((1, _RB, D), lambda t, k: (t, k, 0)),
            pl.BlockSpec((1, D, D), lambda t, k: (t, 0, 0)),
            pl.BlockSpec((1, 1, D), lambda t, k: (t, 0, 0)),
        ],
        out_specs=pl.BlockSpec((2, 1, _RB, HD), lambda t, k: (0, t, k, 0)),
        out_shape=jax.ShapeDtypeStruct((2, 2, N, HD), jnp.float32),
    )(xs, ws, bs)


_node_w_specs = [
    pl.BlockSpec((1, D, D), lambda t, k: (t, 0, 0)),
    pl.BlockSpec((1, 1, D), lambda t, k: (t, 0, 0)),
    pl.BlockSpec((1, D, D), lambda t, k: (t, 0, 0)),
]


def _combine(seg, cnt, xd, wl, bl, wr, swap_xd):
    xmap = ((lambda t, k: (0, 1 - t, k, 0)) if swap_xd
            else (lambda t, k: (0, t, k, 0)))
    return pl.pallas_call(
        _combine_tc,
        grid=(2, N // _RB),
        in_specs=[
            pl.BlockSpec((1, 2, _RB, HD), lambda t, k: (t, 0, k, 0)),
            pl.BlockSpec((1, _RB, 16), lambda t, k: (t, k, 0)),
            pl.BlockSpec((2, 1, _RB, HD), xmap),
        ] + _node_w_specs,
        out_specs=pl.BlockSpec((2, 1, _RB, HD), lambda t, k: (0, t, k, 0)),
        out_shape=jax.ShapeDtypeStruct((2, 2, N, HD), jnp.float32),
    )(seg, cnt, xd, wl, bl, wr)


def _combine2(seg, cnt, xd, wl, bl, wr, w2):
    return pl.pallas_call(
        _combine2_tc,
        grid=(2, N // _RB),
        in_specs=[
            pl.BlockSpec((1, 2, _RB, HD), lambda t, k: (t, 0, k, 0)),
            pl.BlockSpec((1, _RB, 16), lambda t, k: (t, k, 0)),
            pl.BlockSpec((2, 1, _RB, HD), lambda t, k: (0, t, k, 0)),
        ] + _node_w_specs + [
            pl.BlockSpec((1, D, D), lambda t, k: (t, 0, 0)),
        ],
        out_specs=pl.BlockSpec((2, 1, _RB, HD), lambda t, k: (0, t, k, 0)),
        out_shape=jax.ShapeDtypeStruct((2, 2, N, HD), jnp.float32),
    )(seg, cnt, xd, wl, bl, wr, w2)


def _decoder(g, attr, w1c, b1, w2, b2):
    return pl.pallas_call(
        _decoder_tc,
        grid=(E // _EB,),
        in_specs=[
            pl.BlockSpec((2, 2, _EB, HD), lambda k: (0, 0, k, 0)),
            pl.BlockSpec((_EB, 2), lambda k: (k, 0)),
            pl.BlockSpec((2, D), lambda k: (0, 0)),
            pl.BlockSpec((1, D), lambda k: (0, 0)),
            pl.BlockSpec((D, NC), lambda k: (0, 0)),
            pl.BlockSpec((1, NC), lambda k: (0, 0)),
        ],
        out_specs=pl.BlockSpec((_EB, NC), lambda k: (k, 0)),
        out_shape=jax.ShapeDtypeStruct((E, NC), jnp.float32),
    )(g, attr, w1c, b1, w2, b2)


def _tile_idx(a):
    return a.astype(jnp.int32).reshape(NTILES, CH, W)


def _core_idx(a0, a1):
    base = jnp.stack([_tile_idx(a0), _tile_idx(a1)])
    return jnp.stack([base, base + 2 * N])


def kernel(x_politician, x_stock, edge_index_trades, edge_index_rev,
           trade_edge_index, trade_edge_attr, lin_pol_W, lin_pol_b,
           lin_stock_W, lin_stock_b,
           c1t_Wl, c1t_bl, c1t_Wr, c1r_Wl, c1r_bl, c1r_Wr,
           c2t_Wl, c2t_bl, c2t_Wr, c2r_Wl, c2r_bl, c2r_Wr,
           dec_W1, dec_b1, dec_W2, dec_b2):
    f32 = jnp.float32
    src_t = edge_index_trades[0].astype(jnp.int32)
    dst_t = edge_index_trades[1].astype(jnp.int32)
    src_r = edge_index_rev[0].astype(jnp.int32)
    dst_r = edge_index_rev[1].astype(jnp.int32)
    row_d = trade_edge_index[0].astype(jnp.int32)
    col_d = trade_edge_index[1].astype(jnp.int32)

    zeros64 = jnp.zeros((NP, HD), f32)
    zeros16 = jnp.zeros((NP, 16), f32)
    ones16 = jnp.ones((W, 16), f32)

    dst_idx = jnp.stack([_tile_idx(dst_t), _tile_idx(dst_r)])
    src1 = _core_idx(src_t, src_r + N)
    src2 = _core_idx(src_t + N, src_r)
    dec_idx = _core_idx(row_d + N, col_d)

    xs = jnp.stack([x_politician, x_stock])
    ws = jnp.stack([lin_pol_W, lin_stock_W])
    bs = jnp.stack([lin_pol_b, lin_stock_b])[:, None]
    h0 = _proj(xs, ws, bs)

    _seg_counts_kernel, _seg_kernel, _edge_gather_kernel = _sc_kernels()

    seg1, cnt = _seg_counts_kernel(h0.reshape(2 * 2 * N, HD), src1, dst_idx,
                                   zeros64, zeros16, ones16)
    h1 = _combine(seg1, cnt,
                  h0,
                  jnp.stack([c1t_Wl, c1r_Wl]),
                  jnp.stack([c1t_bl, c1r_bl])[:, None],
                  jnp.stack([c1t_Wr, c1r_Wr]),
                  swap_xd=True)

    seg2 = _seg_kernel(h1.reshape(2 * 2 * N, HD), src2, dst_idx, zeros64)
    w1_stock = dec_W1[D:2 * D]
    w1_pol = dec_W1[:D]
    ps = _combine2(seg2, cnt,
                   h1,
                   jnp.stack([c2t_Wl, c2r_Wl]),
                   jnp.stack([c2t_bl, c2r_bl])[:, None],
                   jnp.stack([c2t_Wr, c2r_Wr]),
                   jnp.stack([w1_stock, w1_pol]))

    g = _edge_gather_kernel(ps.reshape(2 * 2 * N, HD), dec_idx)
    w1c = dec_W1[2 * D:]
    return _decoder(g, trade_edge_attr, w1c, dec_b1[None], dec_W2,
                    dec_b2[None])
